# baseline (speedup 1.0000x reference)
import jax
import jax.numpy as jnp
from jax import lax
from jax.experimental import pallas as pl
from jax.experimental.pallas import tpu as pltpu
from jax.experimental.pallas import tpu_sc as plsc

_HID1 = 256
_HID2 = 32
_L = 16
_CH = 64


def _bag_sums_sc(idx_all, table, n_x, n_c):
    info = plsc.get_sparse_core_info()
    nc_, ns_ = info.num_cores, info.num_subcores
    nw = nc_ * ns_
    n_idx = idx_all.shape[1]
    px = n_x // nw
    pc = n_c // nw
    assert n_x % (nw * _CH) == 0 and n_c % (nw * _CH) == 0

    mesh = plsc.VectorSubcoreMesh(core_axis_name="c", subcore_axis_name="s")

    def body(idx_hbm, table_hbm, out_x, out_c, idxb_v, rows_v,
             outbuf_v, gsem, osem):
        wid = lax.axis_index("s") * nc_ + lax.axis_index("c")

        def run_phase(idx_base, out_hbm, out_base, n_bags):
            def chunk_body(c, _):
                cb = c * _CH

                def bag_body(b, _):
                    pltpu.sync_copy(idx_hbm.at[idx_base + cb + b], idxb_v)
                    pltpu.async_copy(table_hbm.at[idxb_v],
                                     rows_v, gsem).wait()

                    def red(r, accs):
                        return tuple(
                            accs[j] + rows_v[r, j // 8,
                                             pl.ds((j % 8) * _L, _L)]
                            for j in range(_HID1 // _L))

                    accs = lax.fori_loop(
                        0, n_idx, red,
                        tuple(jnp.zeros((_L,), jnp.float32)
                              for _ in range(_HID1 // _L)))
                    for j in range(_HID1 // _L):
                        outbuf_v[b, pl.ds(j * _L, _L)] = accs[j]
                    return 0

                lax.fori_loop(0, _CH, bag_body, 0)
                pltpu.async_copy(outbuf_v,
                                 out_hbm.at[pl.ds(out_base + cb, _CH)],
                                 osem).wait()
                return 0

            lax.fori_loop(0, n_bags // _CH, chunk_body, 0)

        run_phase(wid * px, out_x, wid * px, px)
        run_phase(n_x + wid * pc, out_c, wid * pc, pc)

    f = pl.kernel(
        body,
        out_type=(jax.ShapeDtypeStruct((n_x, _HID1), jnp.float32),
                  jax.ShapeDtypeStruct((n_c, _HID1), jnp.float32)),
        mesh=mesh,
        scratch_types=[
            pltpu.VMEM((n_idx,), jnp.int32),
            pltpu.VMEM((n_idx, 2, 128), jnp.float32),
            pltpu.VMEM((_CH, _HID1), jnp.float32),
            pltpu.SemaphoreType.DMA,
            pltpu.SemaphoreType.DMA,
        ],
    )
    return f(idx_all, table.reshape(table.shape[0], 2, 128))


def _mlp_tc(xq, cq, WcT, bc, W2bd, b2t, W3bd, b3t, W4bd, b4t):
    B = xq.shape[0]
    BB = 512
    HB = 4 * _HID1

    def body(xq_r, cq_r, WcT_r, bc_r, W2_r, b2_r, W3_r, b3_r, W4_r, b4_r,
             out_r):
        xe = jnp.maximum(xq_r[...], 0.0)
        c0 = jnp.maximum(cq_r[...], 0.0)
        xs = (xe[:, 0:256] + xe[:, 256:512]
              + xe[:, 512:768] + xe[:, 768:1024])
        cond = c0 + xs
        cond = jnp.maximum(
            jnp.dot(cond, WcT_r[...], preferred_element_type=jnp.float32)
            + bc_r[...], 0.0)
        condt = jnp.concatenate([cond, cond, cond, cond], axis=1)
        h = jnp.maximum(
            jnp.dot(xe, W2_r[...], preferred_element_type=jnp.float32)
            + b2_r[...], 0.0) + condt
        h = jnp.maximum(
            jnp.dot(h, W3_r[...], preferred_element_type=jnp.float32)
            + b3_r[...], 0.0)
        out_r[...] = (
            jnp.dot(h, W4_r[...], preferred_element_type=jnp.float32)
            + b4_r[...])

    def full(a):
        return pl.BlockSpec(a.shape, lambda i: (0,) * a.ndim)

    return pl.pallas_call(
        body,
        grid=(B // BB,),
        in_specs=[
            pl.BlockSpec((BB, HB), lambda i: (i, 0)),
            pl.BlockSpec((BB, _HID1), lambda i: (i, 0)),
            full(WcT), full(bc), full(W2bd), full(b2t), full(W3bd),
            full(b3t), full(W4bd), full(b4t),
        ],
        out_specs=pl.BlockSpec((BB, 4 * 8), lambda i: (i, 0)),
        out_shape=jax.ShapeDtypeStruct((B, 4 * 8), jnp.float32),
    )(xq, cq, WcT, bc, W2bd, b2t, W3bd, b3t, W4bd, b4t)


def kernel(x, condition, embed_weight, Wc, bc, W2, b2, W3, b3, W4, b4):
    B = x.shape[0]
    n_x = B * 4
    n_c = B
    idx_all = jnp.concatenate(
        [x.reshape(n_x, -1), condition], axis=0).astype(jnp.int32)

    out_x, out_c = _bag_sums_sc(idx_all, embed_weight, n_x, n_c)
    xq = out_x.reshape(B, 4 * _HID1)

    bd = jax.scipy.linalg.block_diag
    W2T = W2.T
    W3T = W3.T
    W4Tp = jnp.pad(W4.T, ((0, 0), (0, 3)))
    W2bd = bd(W2T, W2T, W2T, W2T)
    W3bd = bd(W3T, W3T, W3T, W3T)
    W4bd = bd(W4Tp, W4Tp, W4Tp, W4Tp)
    b2t = jnp.tile(b2, 4)[None, :]
    b3t = jnp.tile(b3, 4)[None, :]
    b4t = jnp.tile(jnp.pad(b4, (0, 3)), 4)[None, :]

    out = _mlp_tc(xq, out_c, Wc.T, bc[None, :], W2bd, b2t, W3bd, b3t,
                  W4bd, b4t)
    return out.reshape(B, 4, 8)[..., :5]


# speedup vs baseline: 1.4414x; 1.4414x over previous
import jax
import jax.numpy as jnp
from jax import lax
from jax.experimental import pallas as pl
from jax.experimental.pallas import tpu as pltpu
from jax.experimental.pallas import tpu_sc as plsc

_HID1 = 256
_HID2 = 32
_L = 16
_CH = 64


def _bag_sums_sc(idx_all, table, n_x, n_c):
    info = plsc.get_sparse_core_info()
    nc_, ns_ = info.num_cores, info.num_subcores
    nw = nc_ * ns_
    n_idx = idx_all.shape[1]
    px = n_x // nw
    pc = n_c // nw
    assert n_x % (nw * _CH) == 0 and n_c % (nw * _CH) == 0

    mesh = plsc.VectorSubcoreMesh(core_axis_name="c", subcore_axis_name="s")

    nj = _HID1 // _L

    def body(idx_hbm, table_hbm, out_x, out_c, idx_v, rows_v,
             outbuf_v, gsem0, gsem1, osem):
        wid = lax.axis_index("s") * nc_ + lax.axis_index("c")
        gsems = (gsem0, gsem1)

        def run_phase(idx_base, out_hbm, out_base, n_bags):
            pltpu.sync_copy(idx_hbm.at[pl.ds(idx_base, n_bags)],
                            idx_v.at[pl.ds(0, n_bags)])
            pltpu.async_copy(table_hbm.at[idx_v.at[0]], rows_v.at[0],
                             gsem0)

            def reduce_bag(buf, b):
                def red(r, accs):
                    return tuple(
                        accs[j] + rows_v[buf, r, j // 8,
                                         pl.ds((j % 8) * _L, _L)]
                        for j in range(nj))

                accs = lax.fori_loop(
                    0, n_idx, red,
                    tuple(jnp.zeros((_L,), jnp.float32)
                          for _ in range(nj)))
                slot = lax.rem(b, _CH)
                for j in range(nj):
                    outbuf_v[slot, pl.ds(j * _L, _L)] = accs[j]

            def pair_body(p, _):
                b = 2 * p
                pltpu.async_copy(table_hbm.at[idx_v.at[b + 1]],
                                 rows_v.at[1], gsem1)
                pltpu.make_async_copy(table_hbm.at[idx_v.at[b]],
                                      rows_v.at[0], gsem0).wait()
                reduce_bag(0, b)

                @pl.when(b + 2 < n_bags)
                def _():
                    pltpu.async_copy(table_hbm.at[idx_v.at[b + 2]],
                                     rows_v.at[0], gsem0)

                pltpu.make_async_copy(table_hbm.at[idx_v.at[b + 1]],
                                      rows_v.at[1], gsem1).wait()
                reduce_bag(1, b + 1)

                @pl.when(lax.rem(b + 1, _CH) == _CH - 1)
                def _():
                    start = pl.multiple_of(out_base + b + 2 - _CH, _CH)
                    pltpu.sync_copy(outbuf_v,
                                    out_hbm.at[pl.ds(start, _CH)])

                return 0

            lax.fori_loop(0, n_bags // 2, pair_body, 0)

        run_phase(wid * px, out_x, wid * px, px)
        run_phase(n_x + wid * pc, out_c, wid * pc, pc)

    f = pl.kernel(
        body,
        out_type=(jax.ShapeDtypeStruct((n_x, _HID1), jnp.float32),
                  jax.ShapeDtypeStruct((n_c, _HID1), jnp.float32)),
        mesh=mesh,
        scratch_types=[
            pltpu.VMEM((max(px, pc), n_idx), jnp.int32),
            pltpu.VMEM((2, n_idx, 2, 128), jnp.float32),
            pltpu.VMEM((_CH, _HID1), jnp.float32),
            pltpu.SemaphoreType.DMA,
            pltpu.SemaphoreType.DMA,
            pltpu.SemaphoreType.DMA,
        ],
    )
    return f(idx_all, table.reshape(table.shape[0], 2, 128))


def _mlp_tc(xq, cq, WcT, bc, W2bd, b2t, W3bd, b3t, W4bd, b4t):
    B = xq.shape[0]
    BB = 512
    HB = 4 * _HID1

    def body(xq_r, cq_r, WcT_r, bc_r, W2_r, b2_r, W3_r, b3_r, W4_r, b4_r,
             out_r):
        xe = jnp.maximum(xq_r[...], 0.0)
        c0 = jnp.maximum(cq_r[...], 0.0)
        xs = (xe[:, 0:256] + xe[:, 256:512]
              + xe[:, 512:768] + xe[:, 768:1024])
        cond = c0 + xs
        cond = jnp.maximum(
            jnp.dot(cond, WcT_r[...], preferred_element_type=jnp.float32)
            + bc_r[...], 0.0)
        condt = jnp.concatenate([cond, cond, cond, cond], axis=1)
        h = jnp.maximum(
            jnp.dot(xe, W2_r[...], preferred_element_type=jnp.float32)
            + b2_r[...], 0.0) + condt
        h = jnp.maximum(
            jnp.dot(h, W3_r[...], preferred_element_type=jnp.float32)
            + b3_r[...], 0.0)
        out_r[...] = (
            jnp.dot(h, W4_r[...], preferred_element_type=jnp.float32)
            + b4_r[...])

    def full(a):
        return pl.BlockSpec(a.shape, lambda i: (0,) * a.ndim)

    return pl.pallas_call(
        body,
        grid=(B // BB,),
        in_specs=[
            pl.BlockSpec((BB, HB), lambda i: (i, 0)),
            pl.BlockSpec((BB, _HID1), lambda i: (i, 0)),
            full(WcT), full(bc), full(W2bd), full(b2t), full(W3bd),
            full(b3t), full(W4bd), full(b4t),
        ],
        out_specs=pl.BlockSpec((BB, 4 * 8), lambda i: (i, 0)),
        out_shape=jax.ShapeDtypeStruct((B, 4 * 8), jnp.float32),
    )(xq, cq, WcT, bc, W2bd, b2t, W3bd, b3t, W4bd, b4t)


def kernel(x, condition, embed_weight, Wc, bc, W2, b2, W3, b3, W4, b4):
    B = x.shape[0]
    n_x = B * 4
    n_c = B
    idx_all = jnp.concatenate(
        [x.reshape(n_x, -1), condition], axis=0).astype(jnp.int32)

    out_x, out_c = _bag_sums_sc(idx_all, embed_weight, n_x, n_c)
    xq = out_x.reshape(B, 4 * _HID1)

    bd = jax.scipy.linalg.block_diag
    W2T = W2.T
    W3T = W3.T
    W4Tp = jnp.pad(W4.T, ((0, 0), (0, 3)))
    W2bd = bd(W2T, W2T, W2T, W2T)
    W3bd = bd(W3T, W3T, W3T, W3T)
    W4bd = bd(W4Tp, W4Tp, W4Tp, W4Tp)
    b2t = jnp.tile(b2, 4)[None, :]
    b3t = jnp.tile(b3, 4)[None, :]
    b4t = jnp.tile(jnp.pad(b4, (0, 3)), 4)[None, :]

    out = _mlp_tc(xq, out_c, Wc.T, bc[None, :], W2bd, b2t, W3bd, b3t,
                  W4bd, b4t)
    return out.reshape(B, 4, 8)[..., :5]


# speedup vs baseline: 1.5768x; 1.0940x over previous
import jax
import jax.numpy as jnp
from jax import lax
from jax.experimental import pallas as pl
from jax.experimental.pallas import tpu as pltpu
from jax.experimental.pallas import tpu_sc as plsc

_HID1 = 256
_HID2 = 32
_L = 16
_CH = 64


def _bag_sums_sc(idx_all, table2, n_x, n_c):
    info = plsc.get_sparse_core_info()
    nc_, ns_ = info.num_cores, info.num_subcores
    nw = nc_ * ns_
    n_idx = idx_all.shape[1]
    px = n_x // nw
    pc = n_c // nw
    assert n_x % (nw * _CH) == 0 and n_c % (nw * _CH) == 0

    mesh = plsc.VectorSubcoreMesh(core_axis_name="c", subcore_axis_name="s")

    nj = _HID1 // _L

    def body(idx_hbm, table_hbm, out_x, out_c, idx_v, rows_v,
             outbuf_v, gsem0, gsem1, osem):
        wid = lax.axis_index("s") * nc_ + lax.axis_index("c")
        gsems = (gsem0, gsem1)

        def run_phase(idx_base, out_hbm, out_base, n_bags):
            pltpu.sync_copy(idx_hbm.at[pl.ds(idx_base, n_bags)],
                            idx_v.at[pl.ds(0, n_bags)])
            pltpu.async_copy(table_hbm.at[idx_v.at[0]], rows_v.at[0],
                             gsem0)

            def reduce_bag(buf, b):
                half = n_idx // 2

                def red(r, accs):
                    return tuple(
                        accs[j] + rows_v[buf, r + (j // 8) * half,
                                         pl.ds((j % 8) * _L, _L)]
                        for j in range(nj))

                accs = lax.fori_loop(
                    0, half, red,
                    tuple(jnp.zeros((_L,), jnp.float32)
                          for _ in range(nj)))
                slot = lax.rem(b, _CH)
                for j in range(nj):
                    outbuf_v[slot, pl.ds(j * _L, _L)] = accs[j]

            def pair_body(p, _):
                b = 2 * p
                pltpu.async_copy(table_hbm.at[idx_v.at[b + 1]],
                                 rows_v.at[1], gsem1)
                pltpu.make_async_copy(table_hbm.at[idx_v.at[b]],
                                      rows_v.at[0], gsem0).wait()
                reduce_bag(0, b)

                @pl.when(b + 2 < n_bags)
                def _():
                    pltpu.async_copy(table_hbm.at[idx_v.at[b + 2]],
                                     rows_v.at[0], gsem0)

                pltpu.make_async_copy(table_hbm.at[idx_v.at[b + 1]],
                                      rows_v.at[1], gsem1).wait()
                reduce_bag(1, b + 1)

                @pl.when(lax.rem(b + 1, _CH) == _CH - 1)
                def _():
                    start = pl.multiple_of(out_base + b + 2 - _CH, _CH)
                    pltpu.sync_copy(outbuf_v,
                                    out_hbm.at[pl.ds(start, _CH)])

                return 0

            lax.fori_loop(0, n_bags // 2, pair_body, 0)

        run_phase(wid * px, out_x, wid * px, px)
        run_phase(n_x + wid * pc, out_c, wid * pc, pc)

    f = pl.kernel(
        body,
        out_type=(jax.ShapeDtypeStruct((n_x, _HID1), jnp.float32),
                  jax.ShapeDtypeStruct((n_c, _HID1), jnp.float32)),
        mesh=mesh,
        scratch_types=[
            pltpu.VMEM((max(px, pc), n_idx), jnp.int32),
            pltpu.VMEM((2, n_idx, 128), jnp.float32),
            pltpu.VMEM((_CH, _HID1), jnp.float32),
            pltpu.SemaphoreType.DMA,
            pltpu.SemaphoreType.DMA,
            pltpu.SemaphoreType.DMA,
        ],
    )
    return f(idx_all, table2)


def _mlp_tc(xq, cq, WcT, bc, W2bd, b2t, W3bd, b3t, W4bd, b4t):
    B = xq.shape[0]
    BB = 512
    HB = 4 * _HID1

    def body(xq_r, cq_r, WcT_r, bc_r, W2_r, b2_r, W3_r, b3_r, W4_r, b4_r,
             out_r):
        xe = jnp.maximum(xq_r[...], 0.0)
        c0 = jnp.maximum(cq_r[...], 0.0)
        xs = (xe[:, 0:256] + xe[:, 256:512]
              + xe[:, 512:768] + xe[:, 768:1024])
        cond = c0 + xs
        cond = jnp.maximum(
            jnp.dot(cond, WcT_r[...], preferred_element_type=jnp.float32)
            + bc_r[...], 0.0)
        condt = jnp.concatenate([cond, cond, cond, cond], axis=1)
        h = jnp.maximum(
            jnp.dot(xe, W2_r[...], preferred_element_type=jnp.float32)
            + b2_r[...], 0.0) + condt
        h = jnp.maximum(
            jnp.dot(h, W3_r[...], preferred_element_type=jnp.float32)
            + b3_r[...], 0.0)
        out_r[...] = (
            jnp.dot(h, W4_r[...], preferred_element_type=jnp.float32)
            + b4_r[...])

    def full(a):
        return pl.BlockSpec(a.shape, lambda i: (0,) * a.ndim)

    return pl.pallas_call(
        body,
        grid=(B // BB,),
        in_specs=[
            pl.BlockSpec((BB, HB), lambda i: (i, 0)),
            pl.BlockSpec((BB, _HID1), lambda i: (i, 0)),
            full(WcT), full(bc), full(W2bd), full(b2t), full(W3bd),
            full(b3t), full(W4bd), full(b4t),
        ],
        out_specs=pl.BlockSpec((BB, 4 * 8), lambda i: (i, 0)),
        out_shape=jax.ShapeDtypeStruct((B, 4 * 8), jnp.float32),
    )(xq, cq, WcT, bc, W2bd, b2t, W3bd, b3t, W4bd, b4t)


def kernel(x, condition, embed_weight, Wc, bc, W2, b2, W3, b3, W4, b4):
    B = x.shape[0]
    n_x = B * 4
    n_c = B
    idx = jnp.concatenate(
        [x.reshape(n_x, -1), condition], axis=0).astype(jnp.int32)
    idx_all = jnp.concatenate([2 * idx, 2 * idx + 1], axis=1)
    table2 = embed_weight.reshape(2 * embed_weight.shape[0], 128)

    out_x, out_c = _bag_sums_sc(idx_all, table2, n_x, n_c)
    xq = out_x.reshape(B, 4 * _HID1)

    bd = jax.scipy.linalg.block_diag
    W2T = W2.T
    W3T = W3.T
    W4Tp = jnp.pad(W4.T, ((0, 0), (0, 3)))
    W2bd = bd(W2T, W2T, W2T, W2T)
    W3bd = bd(W3T, W3T, W3T, W3T)
    W4bd = bd(W4Tp, W4Tp, W4Tp, W4Tp)
    b2t = jnp.tile(b2, 4)[None, :]
    b3t = jnp.tile(b3, 4)[None, :]
    b4t = jnp.tile(jnp.pad(b4, (0, 3)), 4)[None, :]

    out = _mlp_tc(xq, out_c, Wc.T, bc[None, :], W2bd, b2t, W3bd, b3t,
                  W4bd, b4t)
    return out.reshape(B, 4, 8)[..., :5]


# speedup vs baseline: 3.9615x; 2.5123x over previous
import jax
import jax.numpy as jnp
from jax import lax
from jax.experimental import pallas as pl
from jax.experimental.pallas import tpu as pltpu
from jax.experimental.pallas import tpu_sc as plsc

_HID1 = 256
_HID2 = 32
_L = 16
_CH = 64


def _bag_sums_sc(idx_all, table, n_x, n_c):
    info = plsc.get_sparse_core_info()
    nc_, ns_ = info.num_cores, info.num_subcores
    nw = nc_ * ns_
    n_idx = idx_all.shape[1]
    px = n_x // nw
    pc = n_c // nw
    assert n_x % (nw * _CH) == 0 and n_c % (nw * _CH) == 0

    mesh = plsc.VectorSubcoreMesh(core_axis_name="c", subcore_axis_name="s")

    nj = _HID1 // _L

    def body(idx_hbm, table_hbm, out_x, out_c, idx_v, rows_v,
             outbuf_v, gsem0, gsem1):
        wid = lax.axis_index("s") * nc_ + lax.axis_index("c")
        h0 = table_hbm.at[:, pl.ds(0, 128)]
        h1 = table_hbm.at[:, pl.ds(128, 128)]
        gsems = (gsem0, gsem1)

        def gather(buf, b):
            pltpu.async_copy(h0.at[idx_v.at[b]], rows_v.at[buf, 0],
                             gsems[buf])
            pltpu.async_copy(h1.at[idx_v.at[b]], rows_v.at[buf, 1],
                             gsems[buf])

        def gwait(buf, b):
            pltpu.make_async_copy(h0.at[idx_v.at[b]], rows_v.at[buf, 0],
                                  gsems[buf]).wait()
            pltpu.make_async_copy(h1.at[idx_v.at[b]], rows_v.at[buf, 1],
                                  gsems[buf]).wait()

        def run_phase(idx_base, out_hbm, out_base, n_bags):
            pltpu.sync_copy(idx_hbm.at[pl.ds(idx_base, n_bags)],
                            idx_v.at[pl.ds(0, n_bags)])
            gather(0, 0)

            def reduce_bag(buf, b):
                def red(r, accs):
                    return tuple(
                        accs[j] + rows_v[buf, j // 8, r,
                                         pl.ds((j % 8) * _L, _L)]
                        for j in range(nj))

                accs = lax.fori_loop(
                    0, n_idx, red,
                    tuple(jnp.zeros((_L,), jnp.float32)
                          for _ in range(nj)))
                slot = lax.rem(b, _CH)
                for j in range(nj):
                    outbuf_v[slot, pl.ds(j * _L, _L)] = accs[j]

            def pair_body(p, _):
                b = 2 * p
                gather(1, b + 1)
                gwait(0, b)
                reduce_bag(0, b)

                @pl.when(b + 2 < n_bags)
                def _():
                    gather(0, b + 2)

                gwait(1, b + 1)
                reduce_bag(1, b + 1)

                @pl.when(lax.rem(b + 1, _CH) == _CH - 1)
                def _():
                    start = pl.multiple_of(out_base + b + 2 - _CH, _CH)
                    pltpu.sync_copy(outbuf_v,
                                    out_hbm.at[pl.ds(start, _CH)])

                return 0

            lax.fori_loop(0, n_bags // 2, pair_body, 0)

        run_phase(wid * px, out_x, wid * px, px)
        run_phase(n_x + wid * pc, out_c, wid * pc, pc)

    f = pl.kernel(
        body,
        out_type=(jax.ShapeDtypeStruct((n_x, _HID1), jnp.float32),
                  jax.ShapeDtypeStruct((n_c, _HID1), jnp.float32)),
        mesh=mesh,
        scratch_types=[
            pltpu.VMEM((max(px, pc), n_idx), jnp.int32),
            pltpu.VMEM((2, 2, n_idx, 128), jnp.float32),
            pltpu.VMEM((_CH, _HID1), jnp.float32),
            pltpu.SemaphoreType.DMA,
            pltpu.SemaphoreType.DMA,
        ],
        compiler_params=pltpu.CompilerParams(use_tc_tiling_on_sc=True),
    )
    return f(idx_all, table)


def _mlp_tc(xq, cq, WcT, bc, W2bd, b2t, W3bd, b3t, W4bd, b4t):
    B = xq.shape[0]
    BB = 512
    HB = 4 * _HID1

    def body(xq_r, cq_r, WcT_r, bc_r, W2_r, b2_r, W3_r, b3_r, W4_r, b4_r,
             out_r):
        xe = jnp.maximum(xq_r[...], 0.0)
        c0 = jnp.maximum(cq_r[...], 0.0)
        xs = (xe[:, 0:256] + xe[:, 256:512]
              + xe[:, 512:768] + xe[:, 768:1024])
        cond = c0 + xs
        cond = jnp.maximum(
            jnp.dot(cond, WcT_r[...], preferred_element_type=jnp.float32)
            + bc_r[...], 0.0)
        condt = jnp.concatenate([cond, cond, cond, cond], axis=1)
        h = jnp.maximum(
            jnp.dot(xe, W2_r[...], preferred_element_type=jnp.float32)
            + b2_r[...], 0.0) + condt
        h = jnp.maximum(
            jnp.dot(h, W3_r[...], preferred_element_type=jnp.float32)
            + b3_r[...], 0.0)
        out_r[...] = (
            jnp.dot(h, W4_r[...], preferred_element_type=jnp.float32)
            + b4_r[...])

    def full(a):
        return pl.BlockSpec(a.shape, lambda i: (0,) * a.ndim)

    return pl.pallas_call(
        body,
        grid=(B // BB,),
        in_specs=[
            pl.BlockSpec((BB, HB), lambda i: (i, 0)),
            pl.BlockSpec((BB, _HID1), lambda i: (i, 0)),
            full(WcT), full(bc), full(W2bd), full(b2t), full(W3bd),
            full(b3t), full(W4bd), full(b4t),
        ],
        out_specs=pl.BlockSpec((BB, 4 * 8), lambda i: (i, 0)),
        out_shape=jax.ShapeDtypeStruct((B, 4 * 8), jnp.float32),
    )(xq, cq, WcT, bc, W2bd, b2t, W3bd, b3t, W4bd, b4t)


def kernel(x, condition, embed_weight, Wc, bc, W2, b2, W3, b3, W4, b4):
    B = x.shape[0]
    n_x = B * 4
    n_c = B
    idx_all = jnp.concatenate(
        [x.reshape(n_x, -1), condition], axis=0).astype(jnp.int32)

    out_x, out_c = _bag_sums_sc(idx_all, embed_weight, n_x, n_c)
    xq = out_x.reshape(B, 4 * _HID1)

    bd = jax.scipy.linalg.block_diag
    W2T = W2.T
    W3T = W3.T
    W4Tp = jnp.pad(W4.T, ((0, 0), (0, 3)))
    W2bd = bd(W2T, W2T, W2T, W2T)
    W3bd = bd(W3T, W3T, W3T, W3T)
    W4bd = bd(W4Tp, W4Tp, W4Tp, W4Tp)
    b2t = jnp.tile(b2, 4)[None, :]
    b3t = jnp.tile(b3, 4)[None, :]
    b4t = jnp.tile(jnp.pad(b4, (0, 3)), 4)[None, :]

    out = _mlp_tc(xq, out_c, Wc.T, bc[None, :], W2bd, b2t, W3bd, b3t,
                  W4bd, b4t)
    return out.reshape(B, 4, 8)[..., :5]


# speedup vs baseline: 3.9747x; 1.0033x over previous
import jax
import jax.numpy as jnp
from jax import lax
from jax.experimental import pallas as pl
from jax.experimental.pallas import tpu as pltpu
from jax.experimental.pallas import tpu_sc as plsc

_HID1 = 256
_HID2 = 32
_L = 16
_CH = 64


def _bag_sums_sc(idx_all, table, n_x, n_c):
    info = plsc.get_sparse_core_info()
    nc_, ns_ = info.num_cores, info.num_subcores
    nw = nc_ * ns_
    n_idx = idx_all.shape[1]
    px = n_x // nw
    pc = n_c // nw
    assert n_x % (nw * _CH) == 0 and n_c % (nw * _CH) == 0

    mesh = plsc.VectorSubcoreMesh(core_axis_name="c", subcore_axis_name="s")

    nj = _HID1 // _L

    def body(idx_hbm, table_hbm, out_x, out_c, idx_v, rows_v,
             outbuf_v, gsem0, gsem1):
        wid = lax.axis_index("s") * nc_ + lax.axis_index("c")
        h0 = table_hbm.at[:, pl.ds(0, 128)]
        h1 = table_hbm.at[:, pl.ds(128, 128)]
        gsems = (gsem0, gsem1)
        nbuf = len(gsems)

        def gather(buf, b):
            pltpu.async_copy(h0.at[idx_v.at[b]], rows_v.at[buf, 0],
                             gsems[buf])
            pltpu.async_copy(h1.at[idx_v.at[b]], rows_v.at[buf, 1],
                             gsems[buf])

        def gwait(buf, b):
            pltpu.make_async_copy(h0.at[idx_v.at[b]], rows_v.at[buf, 0],
                                  gsems[buf]).wait()
            pltpu.make_async_copy(h1.at[idx_v.at[b]], rows_v.at[buf, 1],
                                  gsems[buf]).wait()

        def run_phase(idx_base, out_hbm, out_base, n_bags):
            pltpu.sync_copy(idx_hbm.at[pl.ds(idx_base, n_bags)],
                            idx_v.at[pl.ds(0, n_bags)])
            for k in range(nbuf):
                gather(k, k)

            def reduce_bag(buf, b):
                def red(r, accs):
                    return tuple(
                        accs[j] + rows_v[buf, j // 8, r,
                                         pl.ds((j % 8) * _L, _L)]
                        for j in range(nj))

                accs = lax.fori_loop(
                    0, n_idx, red,
                    tuple(jnp.zeros((_L,), jnp.float32)
                          for _ in range(nj)),
                    unroll=2)
                slot = lax.rem(b, _CH)
                for j in range(nj):
                    outbuf_v[slot, pl.ds(j * _L, _L)] = accs[j]

            def ring_body(q, _):
                b0 = nbuf * q
                for k in range(nbuf):
                    b = b0 + k
                    gwait(k, b)
                    reduce_bag(k, b)

                    @pl.when(b + nbuf < n_bags)
                    def _():
                        gather(k, b + nbuf)

                    @pl.when(lax.rem(b, _CH) == _CH - 1)
                    def _():
                        start = pl.multiple_of(out_base + b + 1 - _CH,
                                               _CH)
                        pltpu.sync_copy(outbuf_v,
                                        out_hbm.at[pl.ds(start, _CH)])

                return 0

            lax.fori_loop(0, n_bags // nbuf, ring_body, 0)

        run_phase(wid * px, out_x, wid * px, px)
        run_phase(n_x + wid * pc, out_c, wid * pc, pc)

    f = pl.kernel(
        body,
        out_type=(jax.ShapeDtypeStruct((n_x, _HID1), jnp.float32),
                  jax.ShapeDtypeStruct((n_c, _HID1), jnp.float32)),
        mesh=mesh,
        scratch_types=[
            pltpu.VMEM((max(px, pc), n_idx), jnp.int32),
            pltpu.VMEM((2, 2, n_idx, 128), jnp.float32),
            pltpu.VMEM((_CH, _HID1), jnp.float32),
            pltpu.SemaphoreType.DMA,
            pltpu.SemaphoreType.DMA,
        ],
        compiler_params=pltpu.CompilerParams(use_tc_tiling_on_sc=True),
    )
    return f(idx_all, table)


def _mlp_tc(xq, cq, WcT, bc, W2bd, b2t, W3bd, b3t, W4bd, b4t):
    B = xq.shape[0]
    BB = 512
    HB = 4 * _HID1

    def body(xq_r, cq_r, WcT_r, bc_r, W2_r, b2_r, W3_r, b3_r, W4_r, b4_r,
             out_r):
        xe = jnp.maximum(xq_r[...], 0.0)
        c0 = jnp.maximum(cq_r[...], 0.0)
        xs = (xe[:, 0:256] + xe[:, 256:512]
              + xe[:, 512:768] + xe[:, 768:1024])
        cond = c0 + xs
        cond = jnp.maximum(
            jnp.dot(cond, WcT_r[...], preferred_element_type=jnp.float32)
            + bc_r[...], 0.0)
        condt = jnp.concatenate([cond, cond, cond, cond], axis=1)
        h = jnp.maximum(
            jnp.dot(xe, W2_r[...], preferred_element_type=jnp.float32)
            + b2_r[...], 0.0) + condt
        h = jnp.maximum(
            jnp.dot(h, W3_r[...], preferred_element_type=jnp.float32)
            + b3_r[...], 0.0)
        out_r[...] = (
            jnp.dot(h, W4_r[...], preferred_element_type=jnp.float32)
            + b4_r[...])

    def full(a):
        return pl.BlockSpec(a.shape, lambda i: (0,) * a.ndim)

    return pl.pallas_call(
        body,
        grid=(B // BB,),
        in_specs=[
            pl.BlockSpec((BB, HB), lambda i: (i, 0)),
            pl.BlockSpec((BB, _HID1), lambda i: (i, 0)),
            full(WcT), full(bc), full(W2bd), full(b2t), full(W3bd),
            full(b3t), full(W4bd), full(b4t),
        ],
        out_specs=pl.BlockSpec((BB, 4 * 8), lambda i: (i, 0)),
        out_shape=jax.ShapeDtypeStruct((B, 4 * 8), jnp.float32),
    )(xq, cq, WcT, bc, W2bd, b2t, W3bd, b3t, W4bd, b4t)


def kernel(x, condition, embed_weight, Wc, bc, W2, b2, W3, b3, W4, b4):
    B = x.shape[0]
    n_x = B * 4
    n_c = B
    idx_all = jnp.concatenate(
        [x.reshape(n_x, -1), condition], axis=0).astype(jnp.int32)

    out_x, out_c = _bag_sums_sc(idx_all, embed_weight, n_x, n_c)
    xq = out_x.reshape(B, 4 * _HID1)

    bd = jax.scipy.linalg.block_diag
    W2T = W2.T
    W3T = W3.T
    W4Tp = jnp.pad(W4.T, ((0, 0), (0, 3)))
    W2bd = bd(W2T, W2T, W2T, W2T)
    W3bd = bd(W3T, W3T, W3T, W3T)
    W4bd = bd(W4Tp, W4Tp, W4Tp, W4Tp)
    b2t = jnp.tile(b2, 4)[None, :]
    b3t = jnp.tile(b3, 4)[None, :]
    b4t = jnp.tile(jnp.pad(b4, (0, 3)), 4)[None, :]

    out = _mlp_tc(xq, out_c, Wc.T, bc[None, :], W2bd, b2t, W3bd, b3t,
                  W4bd, b4t)
    return out.reshape(B, 4, 8)[..., :5]
